# Initial kernel scaffold; baseline (speedup 1.0000x reference)
import jax
import jax.numpy as jnp
from jax import lax
from jax.experimental import pallas as pl
from jax.experimental.pallas import tpu as pltpu
from jax.experimental.pallas import tpu_sc as plsc

B = 4096
L = 200
TOK = B * L
LOC_EMB = 32
TIME_EMB = 32
MODE_EMB = 16
OUT_D = LOC_EMB + TIME_EMB + MODE_EMB

IDXW = 128
NW = 32
ROWS = TOK // IDXW
ROWS_PER_W = ROWS // NW
CHUNK_ROWS = 8
CHUNK_TOK = CHUNK_ROWS * IDXW
N_CHUNKS = ROWS_PER_W // CHUNK_ROWS


def _body(src_hbm, time_hbm, mode_hbm, loc_tab, time_tab, mode_tab, out_hbm,
          sidx, tidx, midx, loc_buf, time_buf, mode_buf, sem):
    cid = lax.axis_index("c")
    sid = lax.axis_index("s")
    wid = sid * 2 + cid
    row0 = wid * ROWS_PER_W

    def chunk(g, carry):
        r = row0 + g * CHUNK_ROWS
        tok0 = r * IDXW
        pltpu.sync_copy(src_hbm.at[pl.ds(r, CHUNK_ROWS)], sidx)
        pltpu.sync_copy(time_hbm.at[pl.ds(r, CHUNK_ROWS)], tidx)
        pltpu.sync_copy(mode_hbm.at[pl.ds(r, CHUNK_ROWS)], midx)
        handles = []
        for j in range(CHUNK_ROWS):
            o = j * IDXW
            handles.append(pltpu.async_copy(
                loc_tab.at[sidx.at[j]], loc_buf.at[pl.ds(o, IDXW)], sem))
            handles.append(pltpu.async_copy(
                time_tab.at[tidx.at[j]], time_buf.at[pl.ds(o, IDXW)], sem))
            handles.append(pltpu.async_copy(
                mode_tab.at[midx.at[j]], mode_buf.at[pl.ds(o, IDXW)], sem))
        for h in handles:
            h.wait()
        pltpu.sync_copy(loc_buf, out_hbm.at[pl.ds(tok0, CHUNK_TOK), pl.ds(0, LOC_EMB)])
        pltpu.sync_copy(time_buf, out_hbm.at[pl.ds(tok0, CHUNK_TOK), pl.ds(LOC_EMB, TIME_EMB)])
        pltpu.sync_copy(mode_buf, out_hbm.at[pl.ds(tok0, CHUNK_TOK), pl.ds(LOC_EMB + TIME_EMB, MODE_EMB)])
        return carry

    lax.fori_loop(0, N_CHUNKS, chunk, 0)


@jax.jit
def _run(src2d, time2d, mode2d, loc_tab, time_tab, mode_tab):
    mesh = plsc.VectorSubcoreMesh(core_axis_name="c", subcore_axis_name="s")
    k = pl.kernel(
        _body,
        out_type=jax.ShapeDtypeStruct((TOK, OUT_D), jnp.float32),
        mesh=mesh,
        scratch_types=[
            pltpu.VMEM((CHUNK_ROWS, IDXW), jnp.int32),
            pltpu.VMEM((CHUNK_ROWS, IDXW), jnp.int32),
            pltpu.VMEM((CHUNK_ROWS, IDXW), jnp.int32),
            pltpu.VMEM((CHUNK_TOK, LOC_EMB), jnp.float32),
            pltpu.VMEM((CHUNK_TOK, TIME_EMB), jnp.float32),
            pltpu.VMEM((CHUNK_TOK, MODE_EMB), jnp.float32),
            pltpu.SemaphoreType.DMA,
        ],
    )
    return k(src2d, time2d, mode2d, loc_tab, time_tab, mode_tab)


def kernel(src, time, mode, emb_loc_table, emb_time_table, emb_mode_table):
    src2d = src.astype(jnp.int32).reshape(ROWS, IDXW)
    time2d = time.astype(jnp.int32).reshape(ROWS, IDXW)
    mode2d = mode.astype(jnp.int32).reshape(ROWS, IDXW)
    out = _run(src2d, time2d, mode2d, emb_loc_table, emb_time_table, emb_mode_table)
    return out.reshape(B, L, OUT_D)


# speedup vs baseline: 1.2259x; 1.2259x over previous
import jax
import jax.numpy as jnp
from jax import lax
from jax.experimental import pallas as pl
from jax.experimental.pallas import tpu as pltpu
from jax.experimental.pallas import tpu_sc as plsc

B = 4096
L = 200
TOK = B * L
LOC_EMB = 32
TIME_EMB = 32
MODE_EMB = 16
OUT_D = LOC_EMB + TIME_EMB + MODE_EMB

IDXW = 128
NW = 32
ROWS = TOK // IDXW
ROWS_PER_W = ROWS // NW
CHUNK_ROWS = 8
CHUNK_TOK = CHUNK_ROWS * IDXW
N_CHUNKS = ROWS_PER_W // CHUNK_ROWS


def _body(src_hbm, time_hbm, mode_hbm, loc_tab, time_tab, mode_tab, out_hbm,
          sidx, tidx, midx, loc_buf, time_buf, mode_buf, sem):
    cid = lax.axis_index("c")
    sid = lax.axis_index("s")
    wid = sid * 2 + cid
    row0 = wid * ROWS_PER_W

    def chunk(g, carry):
        r = row0 + g * CHUNK_ROWS
        tok0 = r * IDXW
        pltpu.sync_copy(src_hbm.at[pl.ds(r, CHUNK_ROWS)], sidx)
        pltpu.sync_copy(time_hbm.at[pl.ds(r, CHUNK_ROWS)], tidx)
        pltpu.sync_copy(mode_hbm.at[pl.ds(r, CHUNK_ROWS)], midx)
        handles = []
        for j in range(CHUNK_ROWS):
            o = j * IDXW
            handles.append(pltpu.async_copy(
                loc_tab.at[sidx.at[j]], loc_buf.at[pl.ds(o, IDXW)], sem))
            handles.append(pltpu.async_copy(
                time_tab.at[tidx.at[j]], time_buf.at[pl.ds(o, IDXW)], sem))
            handles.append(pltpu.async_copy(
                mode_tab.at[midx.at[j]], mode_buf.at[pl.ds(o, IDXW)], sem))
        for h in handles:
            h.wait()
        pltpu.sync_copy(loc_buf, out_hbm.at[pl.ds(tok0, CHUNK_TOK), pl.ds(0, LOC_EMB)])
        pltpu.sync_copy(time_buf, out_hbm.at[pl.ds(tok0, CHUNK_TOK), pl.ds(LOC_EMB, TIME_EMB)])
        pltpu.sync_copy(mode_buf, out_hbm.at[pl.ds(tok0, CHUNK_TOK), pl.ds(LOC_EMB + TIME_EMB, MODE_EMB)])
        return carry

    lax.fori_loop(0, N_CHUNKS, chunk, 0)


@jax.jit
def _run(src2d, time2d, mode2d, loc_tab, time_tab, mode_tab):
    mesh = plsc.VectorSubcoreMesh(core_axis_name="c", subcore_axis_name="s")
    k = pl.kernel(
        _body,
        out_type=jax.ShapeDtypeStruct((TOK, OUT_D), jnp.float32),
        mesh=mesh,
        scratch_types=[
            pltpu.VMEM((CHUNK_ROWS, IDXW), jnp.int32),
            pltpu.VMEM((CHUNK_ROWS, IDXW), jnp.int32),
            pltpu.VMEM((CHUNK_ROWS, IDXW), jnp.int32),
            pltpu.VMEM((CHUNK_TOK, LOC_EMB), jnp.float32),
            pltpu.VMEM((CHUNK_TOK, TIME_EMB), jnp.float32),
            pltpu.VMEM((CHUNK_TOK, MODE_EMB), jnp.float32),
            pltpu.SemaphoreType.DMA,
        ],
        compiler_params=pltpu.CompilerParams(use_tc_tiling_on_sc=False),
    )
    return k(src2d, time2d, mode2d, loc_tab, time_tab, mode_tab)


def kernel(src, time, mode, emb_loc_table, emb_time_table, emb_mode_table):
    src2d = src.astype(jnp.int32).reshape(ROWS, IDXW)
    time2d = time.astype(jnp.int32).reshape(ROWS, IDXW)
    mode2d = mode.astype(jnp.int32).reshape(ROWS, IDXW)
    out = _run(src2d, time2d, mode2d, emb_loc_table, emb_time_table, emb_mode_table)
    return out.reshape(B, L, OUT_D)


# speedup vs baseline: 3.9565x; 3.2273x over previous
import jax
import jax.numpy as jnp
from jax import lax
from jax.experimental import pallas as pl
from jax.experimental.pallas import tpu as pltpu
from jax.experimental.pallas import tpu_sc as plsc

B = 4096
L = 200
TOK = B * L
LOC_EMB = 32
TIME_EMB = 32
MODE_EMB = 16
CMB_EMB = TIME_EMB + MODE_EMB
OUT_D = LOC_EMB + CMB_EMB

IDXW = 128
LANES = 16
NW = 32
TOK_PER_W = TOK // NW
CHUNK = 1024
N_CHUNKS = TOK_PER_W // CHUNK
G_PER_CHUNK = CHUNK // IDXW
V_PER_CHUNK = CHUNK // LANES


def _body(src_hbm, time_hbm, mode_hbm, loc_tab, cmb_tab, out_hbm,
          sidx, tidx, midx, fidx, loc_buf, cmb_buf, sem):
    cid = lax.axis_index("c")
    sid = lax.axis_index("s")
    wid = sid * 2 + cid
    tbase = wid * TOK_PER_W

    def chunk(g, carry):
        tok0 = tbase + g * CHUNK
        pltpu.sync_copy(src_hbm.at[pl.ds(tok0, CHUNK)], sidx)
        pltpu.sync_copy(time_hbm.at[pl.ds(tok0, CHUNK)], tidx)
        pltpu.sync_copy(mode_hbm.at[pl.ds(tok0, CHUNK)], midx)

        def fuse(v, c2):
            o = v * LANES
            t = tidx[pl.ds(o, LANES)]
            m = midx[pl.ds(o, LANES)]
            fidx[pl.ds(o, LANES)] = t * MODE_VOC + m
            return c2

        lax.fori_loop(0, V_PER_CHUNK, fuse, 0)

        handles = []
        for j in range(G_PER_CHUNK):
            o = j * IDXW
            handles.append(pltpu.async_copy(
                loc_tab.at[sidx.at[pl.ds(o, IDXW)]],
                loc_buf.at[pl.ds(o, IDXW)], sem))
            handles.append(pltpu.async_copy(
                cmb_tab.at[fidx.at[pl.ds(o, IDXW)]],
                cmb_buf.at[pl.ds(o, IDXW)], sem))
        for h in handles:
            h.wait()
        pltpu.sync_copy(loc_buf, out_hbm.at[pl.ds(tok0, CHUNK), pl.ds(0, LOC_EMB)])
        pltpu.sync_copy(cmb_buf, out_hbm.at[pl.ds(tok0, CHUNK), pl.ds(LOC_EMB, CMB_EMB)])
        return carry

    lax.fori_loop(0, N_CHUNKS, chunk, 0)


MODE_VOC = 8


@jax.jit
def _run(src1d, time1d, mode1d, loc_tab, cmb_tab):
    mesh = plsc.VectorSubcoreMesh(core_axis_name="c", subcore_axis_name="s")
    k = pl.kernel(
        _body,
        out_type=jax.ShapeDtypeStruct((TOK, OUT_D), jnp.float32),
        mesh=mesh,
        scratch_types=[
            pltpu.VMEM((CHUNK,), jnp.int32),
            pltpu.VMEM((CHUNK,), jnp.int32),
            pltpu.VMEM((CHUNK,), jnp.int32),
            pltpu.VMEM((CHUNK,), jnp.int32),
            pltpu.VMEM((CHUNK, LOC_EMB), jnp.float32),
            pltpu.VMEM((CHUNK, CMB_EMB), jnp.float32),
            pltpu.SemaphoreType.DMA,
        ],
        compiler_params=pltpu.CompilerParams(use_tc_tiling_on_sc=False),
    )
    return k(src1d, time1d, mode1d, loc_tab, cmb_tab)


def kernel(src, time, mode, emb_loc_table, emb_time_table, emb_mode_table):
    cmb_tab = jnp.concatenate(
        [jnp.repeat(emb_time_table, MODE_VOC, axis=0),
         jnp.tile(emb_mode_table, (emb_time_table.shape[0], 1))], axis=-1)
    out = _run(src.astype(jnp.int32).reshape(TOK),
               time.astype(jnp.int32).reshape(TOK),
               mode.astype(jnp.int32).reshape(TOK),
               emb_loc_table, cmb_tab)
    return out.reshape(B, L, OUT_D)
